# Initial kernel scaffold; baseline (speedup 1.0000x reference)
import functools

import jax
import jax.numpy as jnp
from jax import lax
from jax.experimental import pallas as pl
from jax.experimental.pallas import tpu as pltpu
from jax.experimental.pallas import tpu_sc as plsc

N = 10000
E = 320000
IN = 128
H1 = 12
C1 = 32
OUT = 64
B = 64

NP = N + 8
SELFN = N + 240
K = 80
NTILES = 16
R1 = 1000
GRID = N // R1

f32 = jnp.float32
i32 = jnp.int32

_mesh = plsc.VectorSubcoreMesh(core_axis_name="c", subcore_axis_name="s")


def _k1_body(x_ref, w1_ref, as_ref, ad_ref, h1a_ref, h1b_ref, asp_ref, adp_ref):
    h = lax.dot_general(x_ref[...], w1_ref[...], (((1,), (1,)), ((), ())),
                        preferred_element_type=f32)
    h1a_ref[...] = h[:, : 6 * C1]
    h1b_ref[...] = h[:, 6 * C1:]
    asp_ref[...] = jnp.dot(h, as_ref[...], preferred_element_type=f32)
    adp_ref[...] = jnp.dot(h, ad_ref[...], preferred_element_type=f32)


def _k1(x, W1, A_s1, A_d1):
    return pl.pallas_call(
        _k1_body,
        grid=(GRID,),
        in_specs=[
            pl.BlockSpec((R1, IN), lambda i: (i, 0)),
            pl.BlockSpec((H1 * C1, IN), lambda i: (0, 0)),
            pl.BlockSpec((H1 * C1, 16), lambda i: (0, 0)),
            pl.BlockSpec((H1 * C1, 16), lambda i: (0, 0)),
        ],
        out_specs=[
            pl.BlockSpec((R1, 6 * C1), lambda i: (i, 0)),
            pl.BlockSpec((R1, 6 * C1), lambda i: (i, 0)),
            pl.BlockSpec((R1, 16), lambda i: (i, 0)),
            pl.BlockSpec((R1, 16), lambda i: (i, 0)),
        ],
        out_shape=[
            jax.ShapeDtypeStruct((N, 6 * C1), f32),
            jax.ShapeDtypeStruct((N, 6 * C1), f32),
            jax.ShapeDtypeStruct((N, 16), f32),
            jax.ShapeDtypeStruct((N, 16), f32),
        ],
    )(x, W1, A_s1, A_d1)


def _k2_body(as_hbm, ad_hbm, hstk_hbm, self_hbm, esrc_hbm, edst_hbm,
             out_hbm, si_b, di_b, hi_b, as_b, ad_b, h_b, o_b,
             sem_a, sem_b, sem_c):
    c = lax.axis_index("c")
    s = lax.axis_index("s")
    coff = c * NP

    def scoped(acc_ref):
        def run_range(src_hbm, dst_hbm, base, nchunks, add):
            def chunk(i, carry):
                pltpu.sync_copy(src_hbm.at[pl.ds(base + i * K, K)], si_b)
                pltpu.sync_copy(dst_hbm.at[pl.ds(base + i * K, K)], di_b)
                for j in range(K // 16):
                    hi_b[pl.ds(16 * j, 16)] = si_b[pl.ds(16 * j, 16)] + coff
                ca = pltpu.async_copy(as_hbm.at[si_b], as_b, sem_a)
                cb = pltpu.async_copy(ad_hbm.at[di_b], ad_b, sem_b)
                cc = pltpu.async_copy(hstk_hbm.at[hi_b], h_b, sem_c)
                ca.wait()
                cb.wait()
                cc.wait()

                def edge(e, carry2):
                    al = as_b[e, :] + ad_b[e, :]
                    al = jnp.where(al > 0.0, al, al * 0.2)
                    p = jnp.exp(al)
                    o_b[e, 0:16] = p
                    for jh in range(6):
                        ps = o_b[e, 6 * c + jh]
                        spl = jnp.full((16,), ps, dtype=f32)
                        for half in range(2):
                            col = 32 * jh + 16 * half
                            o_b[e, pl.ds(16 + col, 16)] = (
                                h_b[e, pl.ds(col, 16)] * spl)
                    return carry2

                lax.fori_loop(0, K, edge, 0)
                pltpu.sync_copy(o_b, acc_ref.at[di_b], add=add)
                return carry

            lax.fori_loop(0, nchunks, chunk, 0)

        run_range(self_hbm, self_hbm, s * (SELFN // NTILES),
                  SELFN // NTILES // K, False)
        plsc.subcore_barrier()
        run_range(esrc_hbm, edst_hbm, s * (E // NTILES),
                  (E // NTILES) // K, True)
        plsc.subcore_barrier()

        @pl.when(s < 15)
        def _():
            pltpu.sync_copy(acc_ref.at[pl.ds(s * 632, 632)],
                            out_hbm.at[pl.ds(coff + s * 632, 632)])

        @pl.when(s == 15)
        def _():
            pltpu.sync_copy(acc_ref.at[pl.ds(9480, 528)],
                            out_hbm.at[pl.ds(coff + 9480, 528)])

    pl.run_scoped(scoped, pltpu.VMEM_SHARED((NP, 208), f32))


@functools.partial(
    pl.kernel,
    out_type=jax.ShapeDtypeStruct((2 * NP, 208), f32),
    mesh=_mesh,
    scratch_types=[
        pltpu.VMEM((K,), i32),
        pltpu.VMEM((K,), i32),
        pltpu.VMEM((K,), i32),
        pltpu.VMEM((K, 16), f32),
        pltpu.VMEM((K, 16), f32),
        pltpu.VMEM((K, 6 * C1), f32),
        pltpu.VMEM((K, 208), f32),
        pltpu.SemaphoreType.DMA,
        pltpu.SemaphoreType.DMA,
        pltpu.SemaphoreType.DMA,
    ],
)
def _k2(*args):
    _k2_body(*args)


def _k3_body(aa_ref, ab_ref, b1_ref, w2a_ref, w2b_ref, s2_ref, t2_ref,
             da_ref, db_ref, h2_ref, as2_ref, ad2_ref):
    aa = aa_ref[...]
    ab = ab_ref[...]
    dena = jnp.dot(aa, da_ref[...], preferred_element_type=f32)
    denb = jnp.dot(ab, db_ref[...], preferred_element_type=f32)
    h1a = jnp.maximum(aa[:, 16:208] / dena + b1_ref[0:1, : 6 * C1], 0.0)
    h1b = jnp.maximum(ab[:, 16:208] / denb + b1_ref[0:1, 6 * C1:], 0.0)
    h2 = (jnp.dot(h1a, w2a_ref[...], preferred_element_type=f32)
          + jnp.dot(h1b, w2b_ref[...], preferred_element_type=f32))
    h2_ref[...] = h2
    as2_ref[...] = jnp.dot(h2, s2_ref[...], preferred_element_type=f32)
    ad2_ref[...] = jnp.dot(h2, t2_ref[...], preferred_element_type=f32)


def _k3(ACCa, ACCb, B1, W2aT, W2bT, S2, T2, D_a, D_b):
    return pl.pallas_call(
        _k3_body,
        grid=(GRID,),
        in_specs=[
            pl.BlockSpec((R1, 208), lambda i: (i, 0)),
            pl.BlockSpec((R1, 208), lambda i: (i, 0)),
            pl.BlockSpec((8, H1 * C1), lambda i: (0, 0)),
            pl.BlockSpec((6 * C1, OUT), lambda i: (0, 0)),
            pl.BlockSpec((6 * C1, OUT), lambda i: (0, 0)),
            pl.BlockSpec((OUT, 16), lambda i: (0, 0)),
            pl.BlockSpec((OUT, 16), lambda i: (0, 0)),
            pl.BlockSpec((208, 6 * C1), lambda i: (0, 0)),
            pl.BlockSpec((208, 6 * C1), lambda i: (0, 0)),
        ],
        out_specs=[
            pl.BlockSpec((R1, OUT), lambda i: (i, 0)),
            pl.BlockSpec((R1, 16), lambda i: (i, 0)),
            pl.BlockSpec((R1, 16), lambda i: (i, 0)),
        ],
        out_shape=[
            jax.ShapeDtypeStruct((N, OUT), f32),
            jax.ShapeDtypeStruct((N, 16), f32),
            jax.ShapeDtypeStruct((N, 16), f32),
        ],
    )(ACCa, ACCb, B1, W2aT, W2bT, S2, T2, D_a, D_b)


def _k4_body(as_hbm, ad_hbm, h_hbm, self_hbm, esrc_hbm, edst_hbm,
             out_hbm, si_b, di_b, as_b, ad_b, h_b, o_b,
             sem_a, sem_b, sem_c):
    c = lax.axis_index("c")
    s = lax.axis_index("s")
    coff = c * NP
    scale0 = jnp.where(c == 0, 1.0, 0.0).astype(f32)

    def scoped(acc_ref):
        def run_range(src_hbm, dst_hbm, base, nchunks, add, sc):
            def chunk(i, carry):
                pltpu.sync_copy(src_hbm.at[pl.ds(base + i * K, K)], si_b)
                pltpu.sync_copy(dst_hbm.at[pl.ds(base + i * K, K)], di_b)
                ca = pltpu.async_copy(as_hbm.at[si_b], as_b, sem_a)
                cb = pltpu.async_copy(ad_hbm.at[di_b], ad_b, sem_b)
                cc = pltpu.async_copy(h_hbm.at[si_b], h_b, sem_c)
                ca.wait()
                cb.wait()
                cc.wait()

                def edge(e, carry2):
                    al = as_b[e, :] + ad_b[e, :]
                    al = jnp.where(al > 0.0, al, al * 0.2)
                    p = jnp.exp(al) * sc
                    o_b[e, 0:16] = p
                    ps = o_b[e, 0]
                    spl = jnp.full((16,), ps, dtype=f32)
                    for half in range(4):
                        o_b[e, pl.ds(16 + 16 * half, 16)] = (
                            h_b[e, pl.ds(16 * half, 16)] * spl)
                    return carry2

                lax.fori_loop(0, K, edge, 0)
                pltpu.sync_copy(o_b, acc_ref.at[di_b], add=add)
                return carry

            lax.fori_loop(0, nchunks, chunk, 0)

        one = jnp.float32(1.0)
        run_range(self_hbm, self_hbm, s * (SELFN // NTILES),
                  SELFN // NTILES // K, False, scale0)
        plsc.subcore_barrier()
        ebase = c * (E // 2) + s * (E // 2 // NTILES)
        run_range(esrc_hbm, edst_hbm, ebase,
                  (E // 2 // NTILES) // K, True, one)
        plsc.subcore_barrier()

        @pl.when(s < 15)
        def _():
            pltpu.sync_copy(acc_ref.at[pl.ds(s * 632, 632)],
                            out_hbm.at[pl.ds(coff + s * 632, 632)])

        @pl.when(s == 15)
        def _():
            pltpu.sync_copy(acc_ref.at[pl.ds(9480, 528)],
                            out_hbm.at[pl.ds(coff + 9480, 528)])

    pl.run_scoped(scoped, pltpu.VMEM_SHARED((NP, 80), f32))


@functools.partial(
    pl.kernel,
    out_type=jax.ShapeDtypeStruct((2 * NP, 80), f32),
    mesh=_mesh,
    scratch_types=[
        pltpu.VMEM((K,), i32),
        pltpu.VMEM((K,), i32),
        pltpu.VMEM((K, 16), f32),
        pltpu.VMEM((K, 16), f32),
        pltpu.VMEM((K, OUT), f32),
        pltpu.VMEM((K, 80), f32),
        pltpu.SemaphoreType.DMA,
        pltpu.SemaphoreType.DMA,
        pltpu.SemaphoreType.DMA,
    ],
)
def _k4(*args):
    _k4_body(*args)


def _k5_body(a0_ref, a1_ref, bt_ref, b2_ref, d0_ref, out_ref, sums_ref, cnt_ref):
    i = pl.program_id(0)
    sacc = a0_ref[...] + a1_ref[...]
    den = jnp.dot(sacc, d0_ref[...], preferred_element_type=f32)
    h2o = sacc[:, 16:80] / den + b2_ref[0:1, :]
    bt = bt_ref[0, 0, :]
    oh = (bt[:, None] == lax.broadcasted_iota(i32, (R1, B), 1)).astype(f32)
    ps = lax.dot_general(oh, h2o, (((0,), (0,)), ((), ())),
                         preferred_element_type=f32)
    pc = lax.dot_general(oh, jnp.ones((R1, 8), f32), (((0,), (0,)), ((), ())),
                         preferred_element_type=f32)

    @pl.when(i == 0)
    def _():
        sums_ref[...] = ps
        cnt_ref[...] = pc

    @pl.when(i > 0)
    def _():
        sums_ref[...] += ps
        cnt_ref[...] += pc

    @pl.when(i == GRID - 1)
    def _():
        out_ref[...] = sums_ref[...] / jnp.maximum(cnt_ref[:, 0:1], 1.0)


def _k5(A0, A1, batch3, B2, D0):
    return pl.pallas_call(
        _k5_body,
        grid=(GRID,),
        in_specs=[
            pl.BlockSpec((R1, 80), lambda i: (i, 0)),
            pl.BlockSpec((R1, 80), lambda i: (i, 0)),
            pl.BlockSpec((1, 1, R1), lambda i: (i, 0, 0)),
            pl.BlockSpec((8, B), lambda i: (0, 0)),
            pl.BlockSpec((80, B), lambda i: (0, 0)),
        ],
        out_specs=pl.BlockSpec((B, B), lambda i: (0, 0)),
        out_shape=jax.ShapeDtypeStruct((B, B), f32),
        scratch_shapes=[
            pltpu.VMEM((B, B), f32),
            pltpu.VMEM((B, 8), f32),
        ],
    )(A0, A1, batch3, B2, D0)


def kernel(x, edge_index, batch, W1, att_src1, att_dst1, b1,
           W2, att_src2, att_dst2, b2):
    att_s1 = att_src1.reshape(H1, C1)
    att_d1 = att_dst1.reshape(H1, C1)
    eye = jnp.eye(H1, 16, dtype=f32)
    A_s1 = (att_s1[:, :, None] * eye[:, None, :]).reshape(H1 * C1, 16)
    A_d1 = (att_d1[:, :, None] * eye[:, None, :]).reshape(H1 * C1, 16)
    B1 = jnp.tile(b1[None, :], (8, 1))
    W2aT = W2[:, : 6 * C1].T
    W2bT = W2[:, 6 * C1:].T
    e0 = (jnp.arange(16) == 0).astype(f32)
    S2 = att_src2.reshape(OUT, 1) * e0[None, :]
    T2 = att_dst2.reshape(OUT, 1) * e0[None, :]
    ka = jnp.arange(208)[:, None]
    ca = jnp.arange(6 * C1)[None, :] // C1
    D_a = (ka == ca).astype(f32)
    D_b = (ka == ca + 6).astype(f32)
    D0 = (jnp.arange(80)[:, None] == 0).astype(f32) * jnp.ones((1, B), f32)
    B2 = jnp.tile(b2[None, :], (8, 1))
    batch3 = batch.reshape(GRID, 1, R1)

    H1a, H1b, AS1, AD1 = _k1(x, W1, A_s1, A_d1)

    pad = lambda a: jnp.pad(a, ((0, NP - N), (0, 0)))
    HSTK = jnp.concatenate([pad(H1a), pad(H1b)], axis=0)
    SELF = jnp.concatenate([jnp.arange(N, dtype=i32),
                            jnp.full((SELFN - N,), N, dtype=i32)])
    ESRC = edge_index[0]
    EDST = edge_index[1]

    OUT1 = _k2(pad(AS1), pad(AD1), HSTK, SELF, ESRC, EDST)

    H2, AS2, AD2 = _k3(OUT1[0:N], OUT1[NP:NP + N], B1, W2aT, W2bT,
                       S2, T2, D_a, D_b)

    OUT2 = _k4(pad(AS2), pad(AD2), pad(H2), SELF, ESRC, EDST)

    return _k5(OUT2[0:N], OUT2[NP:NP + N], batch3, B2, D0)


# speedup vs baseline: 15.7080x; 15.7080x over previous
import functools

import jax
import jax.numpy as jnp
from jax import lax
from jax.experimental import pallas as pl
from jax.experimental.pallas import tpu as pltpu
from jax.experimental.pallas import tpu_sc as plsc

N = 10000
E = 320000
IN = 128
H1 = 12
C1 = 32
OUT = 64
B = 64

NP = N + 8
SELFN = N + 240
K = 80
NTILES = 16
R1 = 1000
GRID = N // R1

f32 = jnp.float32
i32 = jnp.int32

_sc_cache = {}


def _sc_mesh():
    return plsc.VectorSubcoreMesh(core_axis_name="c", subcore_axis_name="s")


def _k1_body(x_ref, w1_ref, as_ref, ad_ref, h1a_ref, h1b_ref, asp_ref, adp_ref):
    h = lax.dot_general(x_ref[...], w1_ref[...], (((1,), (1,)), ((), ())),
                        preferred_element_type=f32)
    h1a_ref[...] = h[:, : 6 * C1]
    h1b_ref[...] = h[:, 6 * C1:]
    asp_ref[...] = jnp.dot(h, as_ref[...], preferred_element_type=f32)
    adp_ref[...] = jnp.dot(h, ad_ref[...], preferred_element_type=f32)


def _k1(x, W1, A_s1, A_d1):
    return pl.pallas_call(
        _k1_body,
        grid=(GRID,),
        in_specs=[
            pl.BlockSpec((R1, IN), lambda i: (i, 0)),
            pl.BlockSpec((H1 * C1, IN), lambda i: (0, 0)),
            pl.BlockSpec((H1 * C1, 16), lambda i: (0, 0)),
            pl.BlockSpec((H1 * C1, 16), lambda i: (0, 0)),
        ],
        out_specs=[
            pl.BlockSpec((R1, 6 * C1), lambda i: (i, 0)),
            pl.BlockSpec((R1, 6 * C1), lambda i: (i, 0)),
            pl.BlockSpec((R1, 16), lambda i: (i, 0)),
            pl.BlockSpec((R1, 16), lambda i: (i, 0)),
        ],
        out_shape=[
            jax.ShapeDtypeStruct((N, 6 * C1), f32),
            jax.ShapeDtypeStruct((N, 6 * C1), f32),
            jax.ShapeDtypeStruct((N, 16), f32),
            jax.ShapeDtypeStruct((N, 16), f32),
        ],
    )(x, W1, A_s1, A_d1)


def _k2_body(hoff, as_hbm, ad_hbm, hstk_hbm, self_hbm, esrc_hbm, edst_hbm,
             out_hbm, si_b, di_b, hi_b, as_b, ad_b, h_b, o_b, acc_ref,
             sem_a, sem_b, sem_c):
    c = lax.axis_index("c")
    s = lax.axis_index("s")
    coff = c * NP

    def run_range(src_hbm, dst_hbm, base, nchunks, add):
        def chunk(i, carry):
            pltpu.sync_copy(src_hbm.at[pl.ds(base + i * K, K)], si_b)
            pltpu.sync_copy(dst_hbm.at[pl.ds(base + i * K, K)], di_b)
            for j in range(K // 16):
                hi_b[pl.ds(16 * j, 16)] = si_b[pl.ds(16 * j, 16)] + coff
            ca = pltpu.async_copy(as_hbm.at[si_b], as_b, sem_a)
            cb = pltpu.async_copy(ad_hbm.at[di_b], ad_b, sem_b)
            cc = pltpu.async_copy(hstk_hbm.at[hi_b], h_b, sem_c)
            ca.wait()
            cb.wait()
            cc.wait()

            def edge(e, carry2):
                al = as_b[e, :] + ad_b[e, :]
                al = jnp.where(al > 0.0, al, al * 0.2)
                p = jnp.exp(al)
                o_b[e, 0:16] = p
                for jh in range(3):
                    idx = jnp.full((16,), hoff + jh, dtype=i32) + 3 * c
                    spl = p.at[idx].get(mode="promise_in_bounds")
                    for half in range(2):
                        col = 32 * jh + 16 * half
                        o_b[e, pl.ds(16 + col, 16)] = (
                            h_b[e, pl.ds(col, 16)] * spl)
                return carry2

            lax.fori_loop(0, K, edge, 0)
            pltpu.sync_copy(o_b, acc_ref.at[di_b], add=add)
            return carry

        lax.fori_loop(0, nchunks, chunk, 0)

    run_range(self_hbm, self_hbm, s * (SELFN // NTILES),
              SELFN // NTILES // K, False)
    plsc.subcore_barrier()
    run_range(esrc_hbm, edst_hbm, s * (E // NTILES),
              (E // NTILES) // K, True)
    plsc.subcore_barrier()

    @pl.when(s < 15)
    def _():
        pltpu.sync_copy(acc_ref.at[pl.ds(s * 632, 632)],
                        out_hbm.at[pl.ds(coff + s * 632, 632)])

    @pl.when(s == 15)
    def _():
        pltpu.sync_copy(acc_ref.at[pl.ds(9480, 528)],
                        out_hbm.at[pl.ds(coff + 9480, 528)])


def _k2(hoff, *args):
    key = f"k2_{hoff}"
    if key not in _sc_cache:
        _sc_cache[key] = pl.kernel(
            functools.partial(_k2_body, hoff),
            out_type=jax.ShapeDtypeStruct((2 * NP, 112), f32),
            mesh=_sc_mesh(),
            compiler_params=pltpu.CompilerParams(use_tc_tiling_on_sc=False),
            scratch_types=[
                pltpu.VMEM((K,), i32),
                pltpu.VMEM((K,), i32),
                pltpu.VMEM((K,), i32),
                pltpu.VMEM((K, 16), f32),
                pltpu.VMEM((K, 16), f32),
                pltpu.VMEM((K, 3 * C1), f32),
                pltpu.VMEM((K, 112), f32),
                pltpu.VMEM_SHARED((NP, 112), f32),
                pltpu.SemaphoreType.DMA,
                pltpu.SemaphoreType.DMA,
                pltpu.SemaphoreType.DMA,
            ],
        )
    return _sc_cache[key](*args)


def _k3_body(q0_ref, q1_ref, q2_ref, q3_ref, b1_ref,
             w0_ref, w1_ref, w2_ref, w3_ref, s2_ref, t2_ref,
             d0_ref, d1_ref, d2_ref, d3_ref, h2_ref, as2_ref, ad2_ref):
    qs = [q0_ref, q1_ref, q2_ref, q3_ref]
    ds = [d0_ref, d1_ref, d2_ref, d3_ref]
    ws = [w0_ref, w1_ref, w2_ref, w3_ref]
    h2 = None
    for q in range(4):
        a = qs[q][...]
        den = jnp.dot(a, ds[q][...], preferred_element_type=f32)
        h1q = jnp.maximum(a[:, 16:112] / den
                          + b1_ref[0:1, 96 * q:96 * (q + 1)], 0.0)
        part = jnp.dot(h1q, ws[q][...], preferred_element_type=f32)
        h2 = part if h2 is None else h2 + part
    h2_ref[...] = h2
    as2_ref[...] = jnp.dot(h2, s2_ref[...], preferred_element_type=f32)
    ad2_ref[...] = jnp.dot(h2, t2_ref[...], preferred_element_type=f32)


def _k3(quarters, B1, W2qT, S2, T2, Dq):
    return pl.pallas_call(
        _k3_body,
        grid=(GRID,),
        in_specs=(
            [pl.BlockSpec((R1, 112), lambda i: (i, 0))] * 4
            + [pl.BlockSpec((8, H1 * C1), lambda i: (0, 0))]
            + [pl.BlockSpec((96, OUT), lambda i: (0, 0))] * 4
            + [pl.BlockSpec((OUT, 16), lambda i: (0, 0))] * 2
            + [pl.BlockSpec((112, 96), lambda i: (0, 0))] * 4
        ),
        out_specs=[
            pl.BlockSpec((R1, OUT), lambda i: (i, 0)),
            pl.BlockSpec((R1, 16), lambda i: (i, 0)),
            pl.BlockSpec((R1, 16), lambda i: (i, 0)),
        ],
        out_shape=[
            jax.ShapeDtypeStruct((N, OUT), f32),
            jax.ShapeDtypeStruct((N, 16), f32),
            jax.ShapeDtypeStruct((N, 16), f32),
        ],
    )(*quarters, B1, *W2qT, S2, T2, *Dq)


def _k4_body(as_hbm, ad_hbm, h_hbm, self_hbm, esrc_hbm, edst_hbm,
             out_hbm, si_b, di_b, as_b, ad_b, h_b, o_b, acc_ref,
             sem_a, sem_b, sem_c):
    c = lax.axis_index("c")
    s = lax.axis_index("s")
    coff = c * NP
    scale0 = jnp.where(c == 0, 1.0, 0.0).astype(f32)

    if True:
        def run_range(src_hbm, dst_hbm, base, nchunks, add, sc):
            def chunk(i, carry):
                pltpu.sync_copy(src_hbm.at[pl.ds(base + i * K, K)], si_b)
                pltpu.sync_copy(dst_hbm.at[pl.ds(base + i * K, K)], di_b)
                ca = pltpu.async_copy(as_hbm.at[si_b], as_b, sem_a)
                cb = pltpu.async_copy(ad_hbm.at[di_b], ad_b, sem_b)
                cc = pltpu.async_copy(h_hbm.at[si_b], h_b, sem_c)
                ca.wait()
                cb.wait()
                cc.wait()

                def edge(e, carry2):
                    al = as_b[e, :] + ad_b[e, :]
                    al = jnp.where(al > 0.0, al, al * 0.2)
                    p = jnp.exp(al) * sc
                    o_b[e, 0:16] = p
                    spl = p.at[jnp.zeros((16,), i32)].get(
                        mode="promise_in_bounds")
                    for half in range(4):
                        o_b[e, pl.ds(16 + 16 * half, 16)] = (
                            h_b[e, pl.ds(16 * half, 16)] * spl)
                    return carry2

                lax.fori_loop(0, K, edge, 0)
                pltpu.sync_copy(o_b, acc_ref.at[di_b], add=add)
                return carry

            lax.fori_loop(0, nchunks, chunk, 0)

        one = jnp.float32(1.0)
        run_range(self_hbm, self_hbm, s * (SELFN // NTILES),
                  SELFN // NTILES // K, False, scale0)
        plsc.subcore_barrier()
        ebase = c * (E // 2) + s * (E // 2 // NTILES)
        run_range(esrc_hbm, edst_hbm, ebase,
                  (E // 2 // NTILES) // K, True, one)
        plsc.subcore_barrier()

        @pl.when(s < 15)
        def _():
            pltpu.sync_copy(acc_ref.at[pl.ds(s * 632, 632)],
                            out_hbm.at[pl.ds(coff + s * 632, 632)])

        @pl.when(s == 15)
        def _():
            pltpu.sync_copy(acc_ref.at[pl.ds(9480, 528)],
                            out_hbm.at[pl.ds(coff + 9480, 528)])


def _k4(*args):
    if "k4" not in _sc_cache:
        _sc_cache["k4"] = pl.kernel(
            _k4_body,
            out_type=jax.ShapeDtypeStruct((2 * NP, 80), f32),
            mesh=_sc_mesh(),
            compiler_params=pltpu.CompilerParams(use_tc_tiling_on_sc=False),
            scratch_types=[
                pltpu.VMEM((K,), i32),
                pltpu.VMEM((K,), i32),
                pltpu.VMEM((K, 16), f32),
                pltpu.VMEM((K, 16), f32),
                pltpu.VMEM((K, OUT), f32),
                pltpu.VMEM((K, 80), f32),
                pltpu.VMEM_SHARED((NP, 80), f32),
                pltpu.SemaphoreType.DMA,
                pltpu.SemaphoreType.DMA,
                pltpu.SemaphoreType.DMA,
            ],
        )
    return _sc_cache["k4"](*args)


def _k5_body(a0_ref, a1_ref, bt_ref, b2_ref, d0_ref, out_ref, sums_ref, cnt_ref):
    i = pl.program_id(0)
    sacc = a0_ref[...] + a1_ref[...]
    den = jnp.dot(sacc, d0_ref[...], preferred_element_type=f32)
    h2o = sacc[:, 16:80] / den + b2_ref[0:1, :]
    bt = bt_ref[0, 0, :]
    oh = (bt[:, None] == lax.broadcasted_iota(i32, (R1, B), 1)).astype(f32)
    ps = lax.dot_general(oh, h2o, (((0,), (0,)), ((), ())),
                         preferred_element_type=f32)
    pc = lax.dot_general(oh, jnp.ones((R1, 8), f32), (((0,), (0,)), ((), ())),
                         preferred_element_type=f32)

    @pl.when(i == 0)
    def _():
        sums_ref[...] = ps
        cnt_ref[...] = pc

    @pl.when(i > 0)
    def _():
        sums_ref[...] += ps
        cnt_ref[...] += pc

    @pl.when(i == GRID - 1)
    def _():
        out_ref[...] = sums_ref[...] / jnp.maximum(cnt_ref[:, 0:1], 1.0)


def _k5(A0, A1, batch3, B2, D0):
    return pl.pallas_call(
        _k5_body,
        grid=(GRID,),
        in_specs=[
            pl.BlockSpec((R1, 80), lambda i: (i, 0)),
            pl.BlockSpec((R1, 80), lambda i: (i, 0)),
            pl.BlockSpec((1, 1, R1), lambda i: (i, 0, 0)),
            pl.BlockSpec((8, B), lambda i: (0, 0)),
            pl.BlockSpec((80, B), lambda i: (0, 0)),
        ],
        out_specs=pl.BlockSpec((B, B), lambda i: (0, 0)),
        out_shape=jax.ShapeDtypeStruct((B, B), f32),
        scratch_shapes=[
            pltpu.VMEM((B, B), f32),
            pltpu.VMEM((B, 8), f32),
        ],
    )(A0, A1, batch3, B2, D0)


def kernel(x, edge_index, batch, W1, att_src1, att_dst1, b1,
           W2, att_src2, att_dst2, b2):
    att_s1 = att_src1.reshape(H1, C1)
    att_d1 = att_dst1.reshape(H1, C1)
    eye = jnp.eye(H1, 16, dtype=f32)
    A_s1 = (att_s1[:, :, None] * eye[:, None, :]).reshape(H1 * C1, 16)
    A_d1 = (att_d1[:, :, None] * eye[:, None, :]).reshape(H1 * C1, 16)
    B1 = jnp.tile(b1[None, :], (8, 1))
    W2qT = [W2[:, 96 * q:96 * (q + 1)].T for q in range(4)]
    e0 = (jnp.arange(16) == 0).astype(f32)
    S2 = att_src2.reshape(OUT, 1) * e0[None, :]
    T2 = att_dst2.reshape(OUT, 1) * e0[None, :]
    ka = jnp.arange(112)[:, None]
    ca = jnp.arange(96)[None, :] // C1
    Dq = [(ka == ca + 3 * q).astype(f32) for q in range(4)]
    D0 = (jnp.arange(80)[:, None] == 0).astype(f32) * jnp.ones((1, B), f32)
    B2 = jnp.tile(b2[None, :], (8, 1))
    batch3 = batch.reshape(GRID, 1, R1)

    H1a, H1b, AS1, AD1 = _k1(x, W1, A_s1, A_d1)

    pad = lambda a: jnp.pad(a, ((0, NP - N), (0, 0)))
    HSTK1 = jnp.concatenate([pad(H1a[:, :96]), pad(H1a[:, 96:])], axis=0)
    HSTK2 = jnp.concatenate([pad(H1b[:, :96]), pad(H1b[:, 96:])], axis=0)
    SELF = jnp.concatenate([jnp.arange(N, dtype=i32),
                            jnp.full((SELFN - N,), N, dtype=i32)])
    ESRC = edge_index[0]
    EDST = edge_index[1]

    ASp, ADp = pad(AS1), pad(AD1)
    OUTA = _k2(0, ASp, ADp, HSTK1, SELF, ESRC, EDST)
    OUTB = _k2(6, ASp, ADp, HSTK2, SELF, ESRC, EDST)
    quarters = [OUTA[0:N], OUTA[NP:NP + N], OUTB[0:N], OUTB[NP:NP + N]]

    H2, AS2, AD2 = _k3(quarters, B1, W2qT, S2, T2, Dq)

    OUT2 = _k4(pad(AS2), pad(AD2), pad(H2), SELF, ESRC, EDST)

    return _k5(OUT2[0:N], OUT2[NP:NP + N], batch3, B2, D0)
